# Initial kernel scaffold; baseline (speedup 1.0000x reference)
import jax
import jax.numpy as jnp
from jax.experimental import pallas as pl


def kernel(ent_relational_fearues, batch_ent_idxs, relational_adj_matrices, re_ratio, memory_cells, Win_W, gate_W, gate_b, layer_W1, layer_W2):
    raise NotImplementedError("write your pallas kernel here")


# speedup vs baseline: 2.0130x; 2.0130x over previous
import functools

import jax
import jax.numpy as jnp
from jax import lax
from jax.experimental import pallas as pl
from jax.experimental.pallas import tpu as pltpu
from jax.experimental.pallas import tpu_sc as plsc

NUM_ENT = 1000000
ENT_DIM = 128
HIDDEN_DIM = 256
NUM_LAYERS = 2
B = 4096

_NC = 2
_NS = 16
_NW = _NC * _NS
_BPW = B // _NW

@functools.lru_cache(maxsize=None)
def _make_sc_kernels():
    mesh = plsc.VectorSubcoreMesh(
        core_axis_name="c", subcore_axis_name="s", num_cores=_NC, num_subcores=_NS
    )

    @functools.partial(
        pl.kernel,
        mesh=mesh,
        out_type=jax.ShapeDtypeStruct((B, ENT_DIM), jnp.float32),
        scratch_types=[
            pltpu.VMEM((_BPW,), jnp.int32),
            pltpu.VMEM((_BPW, ENT_DIM), jnp.float32),
            pltpu.SemaphoreType.DMA,
        ],
    )
    def sc_gather(table_hbm, idx_hbm, out_hbm, idx_v, rows_v, sem):
        wid = lax.axis_index("s") * _NC + lax.axis_index("c")
        base = wid * _BPW
        pltpu.sync_copy(idx_hbm.at[pl.ds(base, _BPW)], idx_v)
        pltpu.async_copy(table_hbm.at[idx_v], rows_v, sem).wait()
        pltpu.sync_copy(rows_v, out_hbm.at[pl.ds(base, _BPW)])

    @functools.partial(
        pl.kernel,
        mesh=mesh,
        out_type=(),
        scratch_types=[
            pltpu.VMEM((_BPW,), jnp.int32),
            pltpu.VMEM((_BPW, ENT_DIM), jnp.float32),
            pltpu.SemaphoreType.DMA,
        ],
    )
    def sc_scatter(rows_hbm, idx_hbm, table_ref, idx_v, rows_v, sem):
        wid = lax.axis_index("s") * _NC + lax.axis_index("c")
        base = wid * _BPW
        pltpu.sync_copy(idx_hbm.at[pl.ds(base, _BPW)], idx_v)
        pltpu.sync_copy(rows_hbm.at[pl.ds(base, _BPW)], rows_v)
        pltpu.async_copy(rows_v, table_ref.at[idx_v], sem).wait()

    return sc_gather, sc_scatter


def _gate_body(x_ref, win_ref, prev_ref, gw_ref, gb_ref, mult_ref, out_ref):
    x = x_ref[...]
    norm = jnp.maximum(jnp.sum(jnp.abs(x), axis=-1, keepdims=True), 1e-12)
    e = jnp.dot(x / norm, win_ref[...], preferred_element_type=jnp.float32)
    prev = prev_ref[...] * mult_ref[...]
    g = jax.nn.sigmoid(
        jnp.dot(e, gw_ref[0], preferred_element_type=jnp.float32)
        + jnp.dot(prev, gw_ref[1], preferred_element_type=jnp.float32)
        + gb_ref[...]
    )
    out_ref[...] = g * e + (1.0 - g) * prev


def _tc_gate(x, win, prev, gw2, gb, mult):
    return pl.pallas_call(
        _gate_body,
        out_shape=jax.ShapeDtypeStruct((B, ENT_DIM), jnp.float32),
        compiler_params=pltpu.CompilerParams(vmem_limit_bytes=100 * 1024 * 1024),
    )(x, win, prev, gw2, gb, mult)


_BR = 512


def _layer_body(adj_ref, h_ref, w1_ref, w2_ref, out_ref):
    i = pl.program_id(0)
    a = adj_ref[...]
    deg = jnp.sum(a, axis=-1, keepdims=True) + 1e-6
    m = jnp.dot(a, h_ref[...], preferred_element_type=jnp.float32) / deg
    z = jnp.maximum(jnp.dot(m, w1_ref[...], preferred_element_type=jnp.float32), 0.0)
    out_ref[...] = h_ref[pl.ds(i * _BR, _BR), :] + jnp.dot(
        z, w2_ref[...], preferred_element_type=jnp.float32
    )


def _tc_layer(adj, h, w1, w2):
    return pl.pallas_call(
        _layer_body,
        out_shape=jax.ShapeDtypeStruct((B, ENT_DIM), jnp.float32),
        grid=(B // _BR,),
        in_specs=[
            pl.BlockSpec((_BR, B), lambda i: (i, 0)),
            pl.BlockSpec((B, ENT_DIM), lambda i: (0, 0)),
            pl.BlockSpec((ENT_DIM, HIDDEN_DIM), lambda i: (0, 0)),
            pl.BlockSpec((HIDDEN_DIM, ENT_DIM), lambda i: (0, 0)),
        ],
        out_specs=pl.BlockSpec((_BR, ENT_DIM), lambda i: (i, 0)),
        compiler_params=pltpu.CompilerParams(
            dimension_semantics=("arbitrary",),
            vmem_limit_bytes=100 * 1024 * 1024,
        ),
    )(adj, h, w1, w2)


def kernel(ent_relational_fearues, batch_ent_idxs, relational_adj_matrices,
           re_ratio, memory_cells, Win_W, gate_W, gate_b, layer_W1, layer_W2):
    idxs = batch_ent_idxs.astype(jnp.int32)

    n = B
    num_re = jnp.floor(n * jnp.asarray(re_ratio, dtype=jnp.float32)).astype(jnp.int32)
    perm = jax.random.permutation(jax.random.key(42), n)
    keep = jnp.where(jnp.arange(n) < num_re, 0.0, 1.0).astype(jnp.float32)
    row_mult = jnp.ones((n,), dtype=jnp.float32).at[perm].set(keep)
    mult = row_mult[:, None]

    sc_gather, sc_scatter = _make_sc_kernels()
    prev = sc_gather(memory_cells, idxs)

    gw2 = gate_W.reshape(2, ENT_DIM, ENT_DIM)
    memory_out = _tc_gate(ent_relational_fearues, Win_W, prev, gw2, gate_b, mult)

    table_ref = jax.new_ref(memory_cells)
    sc_scatter(memory_out, idxs, table_ref)
    new_memory_cells = jax.freeze(table_ref)

    h = memory_out
    for i in range(NUM_LAYERS):
        h = _tc_layer(relational_adj_matrices, h, layer_W1[i], layer_W2[i])
    return h, new_memory_cells
